# baseline (speedup 1.0000x reference)
import functools

import jax
import jax.numpy as jnp
from jax import lax
from jax.experimental import pallas as pl
from jax.experimental.pallas import tpu as pltpu
from jax.experimental.pallas import tpu_sc as plsc

D = 64
K = 16
CHUNK = 32
FLAT = CHUNK * K
SUB = 512
NC = 2
NS = 16
NW = NC * NS


IDXC = 80


def _mesh():
    return plsc.VectorSubcoreMesh(
        core_axis_name="c", subcore_axis_name="s", num_cores=NC,
        num_subcores=NS)


def _sc_all(item_tab, loc_tab, time_tab, iids, locs, times,
            g_ii, g_ti, g_li, g_il, g_tl, g_it, g_lt):
    nb = iids.shape[0]
    ns = locs.shape[0]
    rows_w = nb // NW
    f32 = jnp.float32
    i32 = jnp.int32

    out_type = tuple(
        jax.ShapeDtypeStruct(s, f32)
        for s in [(nb, D), (nb, D), (nb, D), (nb, D),
                  (ns, D), (ns, D), (ns, D),
                  (ns, D), (ns, D), (ns, D)]
    )
    scratch = [
        pltpu.VMEM((rows_w,), i32),
        pltpu.VMEM((CHUNK,), i32),
        pltpu.VMEM((CHUNK,), i32),
        pltpu.VMEM((IDXC, D), f32),
        pltpu.VMEM((IDXC, D), f32),
        pltpu.VMEM((FLAT,), i32),
        pltpu.VMEM((FLAT,), i32),
        pltpu.VMEM((FLAT,), i32),
        pltpu.VMEM((FLAT,), i32),
        pltpu.VMEM((FLAT, D), f32),
        pltpu.VMEM((FLAT, D), f32),
        pltpu.VMEM((CHUNK, D), f32),
        pltpu.VMEM((CHUNK, D), f32),
        pltpu.SemaphoreType.DMA,
        pltpu.SemaphoreType.DMA,
        pltpu.SemaphoreType.DMA,
        pltpu.SemaphoreType.DMA,
        pltpu.SemaphoreType.DMA,
        pltpu.SemaphoreType.DMA,
    ]

    @functools.partial(
        pl.kernel, out_type=out_type, mesh=_mesh(), scratch_types=scratch,
        compiler_params=pltpu.CompilerParams(use_tc_tiling_on_sc=False))
    def sc(item_t, loc_t, time_t, iids_h, locs_h, times_h,
           hii, hti, hli, hil, htl, hit, hlt,
           o_self_i, o_mii, o_mti, o_mli,
           o_self_l, o_mil, o_mtl,
           o_self_t, o_mit, o_mlt,
           aidx, lidx, tidx, srows0, srows1, fidx0, fidx1, nbr0, nbr1,
           rows0, rows1, mbuf0, mbuf1,
           nsem0, nsem1, esem0, esem1, osem0, osem1):
        wid = lax.axis_index("s") * NC + lax.axis_index("c")
        srows = (srows0, srows1)
        fidx = (fidx0, fidx1)
        nbr = (nbr0, nbr1)
        rows_b = (rows0, rows1)
        mbuf = (mbuf0, mbuf1)
        nsems = (nsem0, nsem1)
        esems = (esem0, esem1)
        osems = (osem0, osem1)

        pltpu.sync_copy(iids_h.at[pl.ds(wid * rows_w, rows_w)], aidx)
        pltpu.sync_copy(locs_h.at[pl.ds(wid * CHUNK, CHUNK)], lidx)
        pltpu.sync_copy(times_h.at[pl.ds(wid * CHUNK, CHUNK)], tidx)

        def reduce_mean(rows, mb):
            def body(c, carry):
                for d in range(D // 16):
                    sl = pl.ds(d * 16, 16)
                    acc = rows[c, sl]
                    for kk in range(1, K):
                        acc = acc + rows[kk * CHUNK + c, sl]
                    mb[c, sl] = acc * (1.0 / K)
                return carry
            lax.fori_loop(0, CHUNK, body, 0)

        def self_job(idx_all, n, tab, out):
            per_w = n // NW
            c = min(IDXC, per_w)
            nch = per_w // c

            def start(j, b):
                pltpu.async_copy(tab.at[idx_all.at[pl.ds(j * c, c)]],
                                 srows[b].at[pl.ds(0, c)], esems[b])

            def finish(j, b):
                pltpu.make_async_copy(tab.at[idx_all.at[pl.ds(j * c, c)]],
                                      srows[b].at[pl.ds(0, c)],
                                      esems[b]).wait()
                pltpu.sync_copy(srows[b].at[pl.ds(0, c)],
                                out.at[pl.ds(wid * per_w + j * c, c)])

            if nch == 1:
                start(0, 0)
                finish(0, 0)
                return
            start(0, 0)
            start(1, 1)

            def outer(jj, carry):
                for b in (0, 1):
                    j = jj * 2 + b
                    finish(j, b)
                    nxt = j + 2

                    @pl.when(nxt < nch)
                    def _():
                        start(nxt, b)
                return carry
            lax.fori_loop(0, nch // 2, outer, 0)

        def mean_job(hgf, mult, idx_all, n, tab, out):
            per_w = n // NW
            nch = per_w // CHUNK

            def start(j, b):
                fx = fidx[b]
                v0 = idx_all[pl.ds(j * CHUNK, 16)]
                v1 = idx_all[pl.ds(j * CHUNK + 16, 16)]
                for kk in range(K):
                    fx[pl.ds(kk * CHUNK, 16)] = v0 + kk * mult
                    fx[pl.ds(kk * CHUNK + 16, 16)] = v1 + kk * mult
                pltpu.async_copy(hgf.at[fx], nbr[b], nsems[b])

            def mid(j, b):
                pltpu.make_async_copy(hgf.at[fidx[b]], nbr[b],
                                      nsems[b]).wait()
                pltpu.async_copy(tab.at[nbr[b]], rows_b[b], esems[b])

            def store_wait(j, b):
                pltpu.make_async_copy(
                    mbuf[b], out.at[pl.ds(wid * per_w + j * CHUNK, CHUNK)],
                    osems[b]).wait()

            def finish(j, b, drain):
                pltpu.make_async_copy(tab.at[nbr[b]], rows_b[b],
                                      esems[b]).wait()

                @pl.when(drain)
                def _():
                    store_wait(j - 2, b)
                reduce_mean(rows_b[b], mbuf[b])
                pltpu.async_copy(
                    mbuf[b], out.at[pl.ds(wid * per_w + j * CHUNK, CHUNK)],
                    osems[b])

            if nch == 1:
                start(0, 0)
                mid(0, 0)
                finish(0, 0, jnp.bool_(False))
                store_wait(0, 0)
                return
            start(0, 0)
            start(1, 1)
            mid(0, 0)

            def outer(jj, carry):
                for b in (0, 1):
                    j = jj * 2 + b
                    pltpu.make_async_copy(tab.at[nbr[b]], rows_b[b],
                                          esems[b]).wait()

                    @pl.when(j >= 2)
                    def _():
                        store_wait(j - 2, b)

                    @pl.when(j + 2 < nch)
                    def _():
                        start(j + 2, b)

                    @pl.when(j + 1 < nch)
                    def _():
                        mid(j + 1, 1 - b)
                    reduce_mean(rows_b[b], mbuf[b])
                    pltpu.async_copy(
                        mbuf[b],
                        out.at[pl.ds(wid * per_w + j * CHUNK, CHUNK)],
                        osems[b])
                return carry
            lax.fori_loop(0, nch // 2, outer, 0)
            store_wait(nch - 2, 0)
            store_wait(nch - 1, 1)

        NI, NL, NT = item_t.shape[0], loc_t.shape[0], time_t.shape[0]
        mean_job(hii, NI, aidx, nb, item_t, o_mii)
        mean_job(hti, NI, aidx, nb, time_t, o_mti)
        mean_job(hli, NI, aidx, nb, loc_t, o_mli)
        self_job(aidx, nb, item_t, o_self_i)
        self_job(lidx, ns, loc_t, o_self_l)
        self_job(tidx, ns, time_t, o_self_t)
        mean_job(hil, NL, lidx, ns, item_t, o_mil)
        mean_job(htl, NL, lidx, ns, time_t, o_mtl)
        mean_job(hit, NT, tidx, ns, item_t, o_mit)
        mean_job(hlt, NT, tidx, ns, loc_t, o_mlt)

    return sc(item_tab, loc_tab, time_tab, iids, locs, times,
              g_ii, g_ti, g_li, g_il, g_tl, g_it, g_lt)


def _att_head(self_rows, aggs, ws, bs, sess, att_w1, att_b1, att_q):
    n, d = self_rows.shape
    n_agg = len(aggs)
    m = n_agg + (1 if sess is not None else 0)
    kx = (1 + n_agg) * d
    hw = n_agg * d
    zw = m * d
    r = min(2048, n)
    g = n // r
    f32 = jnp.float32
    inv_n = 1.0 / n

    wcat = jnp.zeros((kx, hw), f32)
    for p in range(n_agg):
        wcat = wcat.at[0:d, p * d:(p + 1) * d].set(ws[p][:d, :])
        wcat = wcat.at[(1 + p) * d:(2 + p) * d, p * d:(p + 1) * d].set(
            ws[p][d:, :])
    bcat = jnp.concatenate(bs, axis=1)
    w1d = jnp.kron(jnp.eye(m, dtype=f32), att_w1)
    b1d = jnp.tile(att_b1, (1, m))
    qd = jnp.kron(jnp.eye(m, dtype=f32), att_q)

    row64 = pl.BlockSpec((r, d), lambda i: (i, 0))
    rowh = pl.BlockSpec((r, hw), lambda i: (i, 0))
    smem_spec = pl.BlockSpec(memory_space=pltpu.SMEM)

    def full(a):
        return pl.BlockSpec(a.shape, lambda i: tuple(0 for _ in a.shape))

    sess_in = [sess] if sess is not None else []
    sess_specs = [row64] if sess is not None else []

    def body_a(*args):
        i = pl.program_id(0)
        it = iter(args)
        self_b = next(it)[...]
        agg_b = [next(it)[...] for _ in range(n_agg)]
        sess_b = next(it)[...] if sess is not None else None
        wcat_b = next(it)[...]
        bcat_b = next(it)[...]
        w1d_b = next(it)[...]
        b1d_b = next(it)[...]
        qd_b = next(it)[...]
        hcat_ref = next(it)
        wsum_ref = next(it)

        bf16 = jnp.bfloat16
        x = jnp.concatenate([self_b] + agg_b, axis=1)
        h = jnp.maximum(
            jnp.dot(x.astype(bf16), wcat_b.astype(bf16),
                    preferred_element_type=f32) + bcat_b, 0.0)
        hcat_ref[...] = h
        z = h if sess_b is None else jnp.concatenate([h, sess_b], axis=1)
        t = jnp.tanh(
            jnp.dot(z.astype(bf16), w1d_b.astype(bf16),
                    preferred_element_type=f32) + b1d_b)
        w3 = jnp.dot(t, qd_b, preferred_element_type=f32)

        @pl.when(i == 0)
        def _():
            for p in range(m):
                wsum_ref[p] = 0.0

        for p in range(m):
            wsum_ref[p] += jnp.sum(w3[:, p:p + 1])

    hcat, wsum = pl.pallas_call(
        body_a,
        grid=(g,),
        in_specs=[row64] + [row64] * n_agg + sess_specs
                 + [full(wcat), full(bcat), full(w1d), full(b1d), full(qd)],
        out_specs=(rowh, smem_spec),
        out_shape=(jax.ShapeDtypeStruct((n, hw), f32),
                   jax.ShapeDtypeStruct((m,), f32)),
    )(self_rows, *aggs, *sess_in, wcat, bcat, w1d, b1d, qd)

    def body_b(*args):
        it = iter(args)
        hcat_b = next(it)[...]
        sess_b = next(it)[...] if sess is not None else None
        wsum_ref = next(it)
        out_ref = next(it)

        parts = [hcat_b[:, p * d:(p + 1) * d] for p in range(n_agg)]
        if sess_b is not None:
            parts.append(sess_b)
        sv = [jnp.full((1, 1), wsum_ref[p] * inv_n, f32) for p in range(m)]
        mx = sv[0]
        for p in range(1, m):
            mx = jnp.maximum(mx, sv[p])
        ev = [jnp.exp(sv[p] - mx) for p in range(m)]
        denom = ev[0]
        for p in range(1, m):
            denom = denom + ev[p]
        acc = parts[0] * ev[0]
        for p in range(1, m):
            acc = acc + parts[p] * ev[p]
        out_ref[...] = acc / denom

    return pl.pallas_call(
        body_b,
        grid=(g,),
        in_specs=[rowh] + sess_specs + [smem_spec],
        out_specs=row64,
        out_shape=jax.ShapeDtypeStruct((n, d), f32),
    )(hcat, *sess_in, wsum)


def kernel(item, locs, times, session_emb, params, Hg):
    iids = item.reshape(-1)

    def tf(a):
        return a.T.reshape(-1)

    (self_i, mii, mti, mli, self_l, mil, mtl, self_t, mit, mlt) = \
        _sc_all(params['item_tab'], params['loc_tab'],
                params['time_tab'], iids, locs, times,
                tf(Hg['II']), tf(Hg['TI']), tf(Hg['LI']), tf(Hg['IL']),
                tf(Hg['TL']), tf(Hg['IT']), tf(Hg['LT']))
    sess = session_emb[:, 0, :]

    def b2(name):
        return params[name].reshape(1, D)

    h_items = _att_head(
        self_i, [mii, mti, mli],
        [params['II_W'], params['TI_W'], params['LI_W']],
        [b2('II_b'), b2('TI_b'), b2('LI_b')],
        None, params['att_i_W1'], b2('att_i_b1'), params['att_i_q'])
    h_locs = _att_head(
        self_l, [mil, mtl],
        [params['IL_W'], params['TL_W']],
        [b2('IL_b'), b2('TL_b')],
        sess, params['att_l_W1'], b2('att_l_b1'), params['att_l_q'])
    h_times = _att_head(
        self_t, [mit, mlt],
        [params['IT_W'], params['LT_W']],
        [b2('IT_b'), b2('LT_b')],
        sess, params['att_t_W1'], b2('att_t_b1'), params['att_t_q'])
    return (h_items, h_locs, h_times)


# speedup vs baseline: 1.0737x; 1.0737x over previous
import functools

import jax
import jax.numpy as jnp
from jax import lax
from jax.experimental import pallas as pl
from jax.experimental.pallas import tpu as pltpu
from jax.experimental.pallas import tpu_sc as plsc

D = 64
K = 16
CHUNK = 32
FLAT = CHUNK * K
SUB = 512
NC = 2
NS = 16
NW = NC * NS


IDXC = 80


def _mesh():
    return plsc.VectorSubcoreMesh(
        core_axis_name="c", subcore_axis_name="s", num_cores=NC,
        num_subcores=NS)


def _sc_scratch(rows_w):
    f32 = jnp.float32
    i32 = jnp.int32
    return [
        pltpu.VMEM((rows_w,), i32),
        pltpu.VMEM((CHUNK,), i32),
        pltpu.VMEM((CHUNK,), i32),
        pltpu.VMEM((IDXC, D), f32),
        pltpu.VMEM((IDXC, D), f32),
        pltpu.VMEM((FLAT,), i32),
        pltpu.VMEM((FLAT,), i32),
        pltpu.VMEM((FLAT,), i32),
        pltpu.VMEM((FLAT,), i32),
        pltpu.VMEM((FLAT, D), f32),
        pltpu.VMEM((FLAT, D), f32),
        pltpu.VMEM((CHUNK, D), f32),
        pltpu.VMEM((CHUNK, D), f32),
        pltpu.SemaphoreType.DMA,
        pltpu.SemaphoreType.DMA,
        pltpu.SemaphoreType.DMA,
        pltpu.SemaphoreType.DMA,
        pltpu.SemaphoreType.DMA,
        pltpu.SemaphoreType.DMA,
    ]


def _sc_helpers(wid, scr):
    (aidx, lidx, tidx, srows0, srows1, fidx0, fidx1, nbr0, nbr1,
     rows0, rows1, mbuf0, mbuf1,
     nsem0, nsem1, esem0, esem1, osem0, osem1) = scr
    srows = (srows0, srows1)
    fidx = (fidx0, fidx1)
    nbr = (nbr0, nbr1)
    rows_b = (rows0, rows1)
    mbuf = (mbuf0, mbuf1)
    nsems = (nsem0, nsem1)
    esems = (esem0, esem1)
    osems = (osem0, osem1)

    if True:
        def reduce_mean(rows, mb):
            def body(c, carry):
                for d in range(D // 16):
                    sl = pl.ds(d * 16, 16)
                    acc = rows[c, sl]
                    for kk in range(1, K):
                        acc = acc + rows[kk * CHUNK + c, sl]
                    mb[c, sl] = acc * (1.0 / K)
                return carry
            lax.fori_loop(0, CHUNK, body, 0)

        def self_job(idx_all, n, tab, out):
            per_w = n // NW
            c = min(IDXC, per_w)
            nch = per_w // c

            def start(j, b):
                pltpu.async_copy(tab.at[idx_all.at[pl.ds(j * c, c)]],
                                 srows[b].at[pl.ds(0, c)], esems[b])

            def finish(j, b):
                pltpu.make_async_copy(tab.at[idx_all.at[pl.ds(j * c, c)]],
                                      srows[b].at[pl.ds(0, c)],
                                      esems[b]).wait()
                pltpu.sync_copy(srows[b].at[pl.ds(0, c)],
                                out.at[pl.ds(wid * per_w + j * c, c)])

            if nch == 1:
                start(0, 0)
                finish(0, 0)
                return
            start(0, 0)
            start(1, 1)

            def outer(jj, carry):
                for b in (0, 1):
                    j = jj * 2 + b
                    finish(j, b)
                    nxt = j + 2

                    @pl.when(nxt < nch)
                    def _():
                        start(nxt, b)
                return carry
            lax.fori_loop(0, nch // 2, outer, 0)

        def mean_job(hgf, mult, idx_all, n, tab, out):
            per_w = n // NW
            nch = per_w // CHUNK

            def start(j, b):
                fx = fidx[b]
                v0 = idx_all[pl.ds(j * CHUNK, 16)]
                v1 = idx_all[pl.ds(j * CHUNK + 16, 16)]
                for kk in range(K):
                    fx[pl.ds(kk * CHUNK, 16)] = v0 + kk * mult
                    fx[pl.ds(kk * CHUNK + 16, 16)] = v1 + kk * mult
                pltpu.async_copy(hgf.at[fx], nbr[b], nsems[b])

            def mid(j, b):
                pltpu.make_async_copy(hgf.at[fidx[b]], nbr[b],
                                      nsems[b]).wait()
                pltpu.async_copy(tab.at[nbr[b]], rows_b[b], esems[b])

            def store_wait(j, b):
                pltpu.make_async_copy(
                    mbuf[b], out.at[pl.ds(wid * per_w + j * CHUNK, CHUNK)],
                    osems[b]).wait()

            def finish(j, b, drain):
                pltpu.make_async_copy(tab.at[nbr[b]], rows_b[b],
                                      esems[b]).wait()

                @pl.when(drain)
                def _():
                    store_wait(j - 2, b)
                reduce_mean(rows_b[b], mbuf[b])
                pltpu.async_copy(
                    mbuf[b], out.at[pl.ds(wid * per_w + j * CHUNK, CHUNK)],
                    osems[b])

            if nch == 1:
                start(0, 0)
                mid(0, 0)
                finish(0, 0, jnp.bool_(False))
                store_wait(0, 0)
                return
            start(0, 0)
            start(1, 1)
            mid(0, 0)

            def outer(jj, carry):
                for b in (0, 1):
                    j = jj * 2 + b
                    pltpu.make_async_copy(tab.at[nbr[b]], rows_b[b],
                                          esems[b]).wait()

                    @pl.when(j >= 2)
                    def _():
                        store_wait(j - 2, b)

                    @pl.when(j + 2 < nch)
                    def _():
                        start(j + 2, b)

                    @pl.when(j + 1 < nch)
                    def _():
                        mid(j + 1, 1 - b)
                    reduce_mean(rows_b[b], mbuf[b])
                    pltpu.async_copy(
                        mbuf[b],
                        out.at[pl.ds(wid * per_w + j * CHUNK, CHUNK)],
                        osems[b])
                return carry
            lax.fori_loop(0, nch // 2, outer, 0)
            store_wait(nch - 2, 0)
            store_wait(nch - 1, 1)

    def preload(iids_h, locs_h, times_h, rows_w):
        pltpu.sync_copy(iids_h.at[pl.ds(wid * rows_w, rows_w)], aidx)
        pltpu.sync_copy(locs_h.at[pl.ds(wid * CHUNK, CHUNK)], lidx)
        pltpu.sync_copy(times_h.at[pl.ds(wid * CHUNK, CHUNK)], tidx)

    return preload, self_job, mean_job, aidx, lidx, tidx


def _sc_part1(loc_tab, time_tab, iids, locs, times, g_ti, g_li, g_tl, g_lt):
    nb = iids.shape[0]
    ns = locs.shape[0]
    rows_w = nb // NW
    f32 = jnp.float32
    out_type = tuple(
        jax.ShapeDtypeStruct(s, f32)
        for s in [(nb, D), (nb, D), (ns, D), (ns, D), (ns, D), (ns, D)])

    @functools.partial(
        pl.kernel, out_type=out_type, mesh=_mesh(),
        scratch_types=_sc_scratch(rows_w),
        compiler_params=pltpu.CompilerParams(use_tc_tiling_on_sc=False))
    def sc(loc_t, time_t, iids_h, locs_h, times_h, hti, hli, htl, hlt,
           o_mti, o_mli, o_self_l, o_mtl, o_self_t, o_mlt, *scr):
        wid = lax.axis_index("s") * NC + lax.axis_index("c")
        preload, self_job, mean_job, aidx, lidx, tidx = _sc_helpers(wid, scr)
        preload(iids_h, locs_h, times_h, rows_w)
        ni = hti.shape[0] // K
        nl = htl.shape[0] // K
        nt = hlt.shape[0] // K
        mean_job(hti, ni, aidx, nb, time_t, o_mti)
        mean_job(hli, ni, aidx, nb, loc_t, o_mli)
        self_job(lidx, ns, loc_t, o_self_l)
        mean_job(htl, nl, lidx, ns, time_t, o_mtl)
        self_job(tidx, ns, time_t, o_self_t)
        mean_job(hlt, nt, tidx, ns, loc_t, o_mlt)

    return sc(loc_tab, time_tab, iids, locs, times, g_ti, g_li, g_tl, g_lt)


def _sc_part2(item_tab, iids, locs, times, g_ii, g_il, g_it):
    nb = iids.shape[0]
    ns = locs.shape[0]
    rows_w = nb // NW
    f32 = jnp.float32
    out_type = tuple(
        jax.ShapeDtypeStruct(s, f32)
        for s in [(nb, D), (nb, D), (ns, D), (ns, D)])

    @functools.partial(
        pl.kernel, out_type=out_type, mesh=_mesh(),
        scratch_types=_sc_scratch(rows_w),
        compiler_params=pltpu.CompilerParams(use_tc_tiling_on_sc=False))
    def sc(item_t, iids_h, locs_h, times_h, hii, hil, hit,
           o_self_i, o_mii, o_mil, o_mit, *scr):
        wid = lax.axis_index("s") * NC + lax.axis_index("c")
        preload, self_job, mean_job, aidx, lidx, tidx = _sc_helpers(wid, scr)
        preload(iids_h, locs_h, times_h, rows_w)
        ni = hii.shape[0] // K
        nl = hil.shape[0] // K
        nt = hit.shape[0] // K
        mean_job(hii, ni, aidx, nb, item_t, o_mii)
        self_job(aidx, nb, item_t, o_self_i)
        mean_job(hil, nl, lidx, ns, item_t, o_mil)
        mean_job(hit, nt, tidx, ns, item_t, o_mit)

    return sc(item_tab, iids, locs, times, g_ii, g_il, g_it)


def _att_head(self_rows, aggs, ws, bs, sess, att_w1, att_b1, att_q):
    n, d = self_rows.shape
    n_agg = len(aggs)
    m = n_agg + (1 if sess is not None else 0)
    kx = (1 + n_agg) * d
    hw = n_agg * d
    zw = m * d
    r = min(2048, n)
    g = n // r
    f32 = jnp.float32
    inv_n = 1.0 / n

    wcat = jnp.zeros((kx, hw), f32)
    for p in range(n_agg):
        wcat = wcat.at[0:d, p * d:(p + 1) * d].set(ws[p][:d, :])
        wcat = wcat.at[(1 + p) * d:(2 + p) * d, p * d:(p + 1) * d].set(
            ws[p][d:, :])
    bcat = jnp.concatenate(bs, axis=1)
    w1d = jnp.kron(jnp.eye(m, dtype=f32), att_w1)
    b1d = jnp.tile(att_b1, (1, m))
    qd = jnp.kron(jnp.eye(m, dtype=f32), att_q)

    row64 = pl.BlockSpec((r, d), lambda i: (i, 0))
    rowh = pl.BlockSpec((r, hw), lambda i: (i, 0))
    smem_spec = pl.BlockSpec(memory_space=pltpu.SMEM)

    def full(a):
        return pl.BlockSpec(a.shape, lambda i: tuple(0 for _ in a.shape))

    sess_in = [sess] if sess is not None else []
    sess_specs = [row64] if sess is not None else []

    def body_a(*args):
        i = pl.program_id(0)
        it = iter(args)
        self_b = next(it)[...]
        agg_b = [next(it)[...] for _ in range(n_agg)]
        sess_b = next(it)[...] if sess is not None else None
        wcat_b = next(it)[...]
        bcat_b = next(it)[...]
        w1d_b = next(it)[...]
        b1d_b = next(it)[...]
        qd_b = next(it)[...]
        hcat_ref = next(it)
        wsum_ref = next(it)

        x = jnp.concatenate([self_b] + agg_b, axis=1)
        h = jnp.maximum(
            jnp.dot(x, wcat_b, preferred_element_type=f32) + bcat_b, 0.0)
        hcat_ref[...] = h
        z = h if sess_b is None else jnp.concatenate([h, sess_b], axis=1)
        t = jnp.tanh(jnp.dot(z, w1d_b, preferred_element_type=f32) + b1d_b)
        w3 = jnp.dot(t, qd_b, preferred_element_type=f32)

        @pl.when(i == 0)
        def _():
            for p in range(m):
                wsum_ref[p] = 0.0

        for p in range(m):
            wsum_ref[p] += jnp.sum(w3[:, p:p + 1])

    hcat, wsum = pl.pallas_call(
        body_a,
        grid=(g,),
        in_specs=[row64] + [row64] * n_agg + sess_specs
                 + [full(wcat), full(bcat), full(w1d), full(b1d), full(qd)],
        out_specs=(rowh, smem_spec),
        out_shape=(jax.ShapeDtypeStruct((n, hw), f32),
                   jax.ShapeDtypeStruct((m,), f32)),
    )(self_rows, *aggs, *sess_in, wcat, bcat, w1d, b1d, qd)

    def body_b(*args):
        it = iter(args)
        hcat_b = next(it)[...]
        sess_b = next(it)[...] if sess is not None else None
        wsum_ref = next(it)
        out_ref = next(it)

        parts = [hcat_b[:, p * d:(p + 1) * d] for p in range(n_agg)]
        if sess_b is not None:
            parts.append(sess_b)
        sv = [jnp.full((1, 1), wsum_ref[p] * inv_n, f32) for p in range(m)]
        mx = sv[0]
        for p in range(1, m):
            mx = jnp.maximum(mx, sv[p])
        ev = [jnp.exp(sv[p] - mx) for p in range(m)]
        denom = ev[0]
        for p in range(1, m):
            denom = denom + ev[p]
        acc = parts[0] * ev[0]
        for p in range(1, m):
            acc = acc + parts[p] * ev[p]
        out_ref[...] = acc / denom

    return pl.pallas_call(
        body_b,
        grid=(g,),
        in_specs=[rowh] + sess_specs + [smem_spec],
        out_specs=row64,
        out_shape=jax.ShapeDtypeStruct((n, d), f32),
    )(hcat, *sess_in, wsum)


def kernel(item, locs, times, session_emb, params, Hg):
    iids = item.reshape(-1)

    def tf(a):
        return a.T.reshape(-1)

    (mti, mli, self_l, mtl, self_t, mlt) = _sc_part1(
        params['loc_tab'], params['time_tab'], iids, locs, times,
        tf(Hg['TI']), tf(Hg['LI']), tf(Hg['TL']), tf(Hg['LT']))
    (self_i, mii, mil, mit) = _sc_part2(
        params['item_tab'], iids, locs, times,
        tf(Hg['II']), tf(Hg['IL']), tf(Hg['IT']))
    sess = session_emb[:, 0, :]

    def b2(name):
        return params[name].reshape(1, D)

    h_items = _att_head(
        self_i, [mii, mti, mli],
        [params['II_W'], params['TI_W'], params['LI_W']],
        [b2('II_b'), b2('TI_b'), b2('LI_b')],
        None, params['att_i_W1'], b2('att_i_b1'), params['att_i_q'])
    h_locs = _att_head(
        self_l, [mil, mtl],
        [params['IL_W'], params['TL_W']],
        [b2('IL_b'), b2('TL_b')],
        sess, params['att_l_W1'], b2('att_l_b1'), params['att_l_q'])
    h_times = _att_head(
        self_t, [mit, mlt],
        [params['IT_W'], params['LT_W']],
        [b2('IT_b'), b2('LT_b')],
        sess, params['att_t_W1'], b2('att_t_b1'), params['att_t_q'])
    return (h_items, h_locs, h_times)
